# baseline (speedup 1.0000x reference)
import functools

import jax
import jax.numpy as jnp
from jax.experimental import pallas as pl
from jax.experimental.pallas import tpu as pltpu

_NBUF = 5


def _ring_body(n, block_rows, num_blocks, x_ref, w_ref, a_hbm, o_ref, support_ref, bufs, sems):
    r = pl.program_id(0)

    @pl.when(r == 0)
    def _prime():
        support_ref[...] = jnp.dot(
            x_ref[...], w_ref[...], preferred_element_type=jnp.float32
        )
        for b in range(_NBUF):
            pltpu.make_async_copy(
                a_hbm.at[pl.ds(b * block_rows, block_rows), :], bufs.at[b], sems.at[b]
            ).start()

    for b in range(_NBUF):
        i = r * _NBUF + b
        pltpu.make_async_copy(
            a_hbm.at[pl.ds(i * block_rows, block_rows), :], bufs.at[b], sems.at[b]
        ).wait()
        o_ref[b * block_rows : (b + 1) * block_rows, :] = jnp.dot(
            bufs[b], support_ref[...], preferred_element_type=jnp.float32
        )

        @pl.when(i + _NBUF < num_blocks)
        def _refill():
            pltpu.make_async_copy(
                a_hbm.at[pl.ds((i + _NBUF) * block_rows, block_rows), :],
                bufs.at[b],
                sems.at[b],
            ).start()


def _fused_body(x_ref, w_ref, a_ref, o_ref, support_ref):
    @pl.when(pl.program_id(0) == 0)
    def _():
        support_ref[...] = jnp.dot(
            x_ref[...], w_ref[...], preferred_element_type=jnp.float32
        )

    o_ref[...] = jnp.dot(
        a_ref[...], support_ref[...], preferred_element_type=jnp.float32
    )


@functools.partial(jax.jit, static_argnames=("block_rows",))
def _graph_conv_ring(input_tensor, adj_mat, weights, block_rows=200):
    n, f_in = input_tensor.shape
    f_out = weights.shape[1]
    num_blocks = n // block_rows
    grid = num_blocks // _NBUF
    body = functools.partial(_ring_body, n, block_rows, num_blocks)
    return pl.pallas_call(
        body,
        grid=(grid,),
        in_specs=[
            pl.BlockSpec((n, f_in), lambda r: (0, 0)),
            pl.BlockSpec((f_in, f_out), lambda r: (0, 0)),
            pl.BlockSpec(memory_space=pl.ANY),
        ],
        out_specs=pl.BlockSpec((block_rows * _NBUF, f_out), lambda r: (r, 0)),
        out_shape=jax.ShapeDtypeStruct((n, f_out), jnp.float32),
        scratch_shapes=[
            pltpu.VMEM((n, f_out), jnp.float32),
            pltpu.VMEM((_NBUF, block_rows, n), jnp.float32),
            pltpu.SemaphoreType.DMA((_NBUF,)),
        ],
        compiler_params=pltpu.CompilerParams(
            dimension_semantics=("arbitrary",),
        ),
    )(input_tensor, weights, adj_mat)


@functools.partial(jax.jit, static_argnames=("block_rows",))
def _graph_conv_fused(input_tensor, adj_mat, weights, block_rows=400):
    n, f_in = input_tensor.shape
    f_out = weights.shape[1]
    grid = pl.cdiv(n, block_rows)
    return pl.pallas_call(
        _fused_body,
        grid=(grid,),
        in_specs=[
            pl.BlockSpec((n, f_in), lambda i: (0, 0)),
            pl.BlockSpec((f_in, f_out), lambda i: (0, 0)),
            pl.BlockSpec((block_rows, n), lambda i: (i, 0)),
        ],
        out_specs=pl.BlockSpec((block_rows, f_out), lambda i: (i, 0)),
        out_shape=jax.ShapeDtypeStruct((n, f_out), jnp.float32),
        scratch_shapes=[pltpu.VMEM((n, f_out), jnp.float32)],
        compiler_params=pltpu.CompilerParams(
            dimension_semantics=("arbitrary",),
        ),
    )(input_tensor, weights, adj_mat)


def kernel(input_tensor, adj_mat, kernel):
    n = input_tensor.shape[0]
    if n % (200 * _NBUF) == 0:
        return _graph_conv_ring(input_tensor, adj_mat, kernel, block_rows=200)
    return _graph_conv_fused(input_tensor, adj_mat, kernel)


# speedup vs baseline: 1.0335x; 1.0335x over previous
import functools

import jax
import jax.numpy as jnp
from jax.experimental import pallas as pl
from jax.experimental.pallas import tpu as pltpu


def _body(x_ref, w_ref, a0_ref, a1_ref, o_ref, support_ref):
    @pl.when(pl.program_id(0) == 0)
    def _():
        support_ref[...] = jnp.dot(
            x_ref[...], w_ref[...], preferred_element_type=jnp.float32
        )

    half = a0_ref.shape[0]
    o_ref[:half, :] = jnp.dot(
        a0_ref[...], support_ref[...], preferred_element_type=jnp.float32
    )
    o_ref[half:, :] = jnp.dot(
        a1_ref[...], support_ref[...], preferred_element_type=jnp.float32
    )


@functools.partial(jax.jit, static_argnames=("block_rows",))
def _graph_conv(input_tensor, adj_mat, weights, block_rows=400):
    n, f_in = input_tensor.shape
    f_out = weights.shape[1]
    half = block_rows // 2
    grid = pl.cdiv(n, block_rows)
    return pl.pallas_call(
        _body,
        grid=(grid,),
        in_specs=[
            pl.BlockSpec((n, f_in), lambda i: (0, 0)),
            pl.BlockSpec((f_in, f_out), lambda i: (0, 0)),
            pl.BlockSpec((half, n), lambda i: (2 * i, 0)),
            pl.BlockSpec((half, n), lambda i: (2 * i + 1, 0)),
        ],
        out_specs=pl.BlockSpec((block_rows, f_out), lambda i: (i, 0)),
        out_shape=jax.ShapeDtypeStruct((n, f_out), jnp.float32),
        scratch_shapes=[pltpu.VMEM((n, f_out), jnp.float32)],
        compiler_params=pltpu.CompilerParams(
            dimension_semantics=("arbitrary",),
        ),
    )(input_tensor, weights, adj_mat, adj_mat)


def kernel(input_tensor, adj_mat, kernel):
    return _graph_conv(input_tensor, adj_mat, kernel)


# speedup vs baseline: 1.0388x; 1.0052x over previous
import functools

import jax
import jax.numpy as jnp
from jax.experimental import pallas as pl
from jax.experimental.pallas import tpu as pltpu


def _body(x_ref, w_ref, a_ref, o_ref, support_ref):
    @pl.when(pl.program_id(0) == 0)
    def _():
        support_ref[...] = jnp.dot(
            x_ref[...], w_ref[...], preferred_element_type=jnp.float32
        ).astype(jnp.bfloat16)

    o_ref[...] = jnp.dot(
        a_ref[...].astype(jnp.bfloat16),
        support_ref[...],
        preferred_element_type=jnp.float32,
    )


@functools.partial(jax.jit, static_argnames=("block_rows",))
def _graph_conv(input_tensor, adj_mat, weights, block_rows=400):
    n, f_in = input_tensor.shape
    f_out = weights.shape[1]
    grid = pl.cdiv(n, block_rows)
    return pl.pallas_call(
        _body,
        grid=(grid,),
        in_specs=[
            pl.BlockSpec((n, f_in), lambda i: (0, 0)),
            pl.BlockSpec((f_in, f_out), lambda i: (0, 0)),
            pl.BlockSpec((block_rows, n), lambda i: (i, 0)),
        ],
        out_specs=pl.BlockSpec((block_rows, f_out), lambda i: (i, 0)),
        out_shape=jax.ShapeDtypeStruct((n, f_out), jnp.float32),
        scratch_shapes=[pltpu.VMEM((n, f_out), jnp.bfloat16)],
        compiler_params=pltpu.CompilerParams(
            dimension_semantics=("arbitrary",),
        ),
    )(input_tensor, weights, adj_mat)


def kernel(input_tensor, adj_mat, kernel):
    return _graph_conv(input_tensor, adj_mat, kernel)
